# baseline (speedup 1.0000x reference)
import jax
import jax.numpy as jnp
from jax.experimental import pallas as pl
from jax.experimental.pallas import tpu as pltpu


def _head_body(pooled_ref, Wf1_ref, bf1_ref, bn_ref, Wf2_ref, bf2_ref, out_ref):
    f = jnp.maximum(pooled_ref[...] @ Wf1_ref[...] + bf1_ref[...], 0.0)
    g, b, m, v = (bn_ref[0:1, :], bn_ref[1:2, :], bn_ref[2:3, :], bn_ref[3:4, :])
    f = (f - m) / jnp.sqrt(v + 1e-5) * g + b
    out_ref[...] = f @ Wf2_ref[...] + bf2_ref[...]


def _conv(x, src, dst, ea, Wq, bq, Wk, bk, Wv, bv, We, Ws, bs):
    q = x @ Wq + bq
    k = x @ Wk + bk
    v = x @ Wv + bv
    e = ea @ We
    scale = 1.0 / jnp.sqrt(jnp.asarray(q.shape[-1], jnp.float32))
    logits = jnp.sum(q[dst] * (k[src] + e), axis=-1) * scale
    m = jax.ops.segment_max(logits, dst, num_segments=10000)
    m = jnp.where(jnp.isfinite(m), m, 0.0)
    ex = jnp.exp(logits - m[dst])
    den = jax.ops.segment_sum(ex, dst, num_segments=10000)
    alpha = ex / (den[dst] + 1e-16)
    agg = jax.ops.segment_sum(alpha[:, None] * (v[src] + e), dst, num_segments=10000)
    return agg + x @ Ws + bs


def kernel(x, edge_index, edge_attr, batch, Wq1, bq1, Wk1, bk1, Wv1, bv1, We1, Ws1, bs1, Wq2, bq2, Wk2, bk2, Wv2, bv2, We2, Ws2, bs2, Wf1, bf1, bn_g, bn_b, bn_m, bn_v, Wf2, bf2):
    src, dst = edge_index[0], edge_index[1]
    h = jax.nn.relu(_conv(x, src, dst, edge_attr, Wq1, bq1, Wk1, bk1, Wv1, bv1, We1, Ws1, bs1))
    h = jax.nn.relu(_conv(h, src, dst, edge_attr, Wq2, bq2, Wk2, bk2, Wv2, bv2, We2, Ws2, bs2))
    pooled = jax.ops.segment_max(h, batch, num_segments=64)
    pooled = jnp.where(jnp.isfinite(pooled), pooled, 0.0)
    bn = jnp.stack([bn_g, bn_b, bn_m, bn_v], axis=0)
    out = pl.pallas_call(
        _head_body,
        out_shape=jax.ShapeDtypeStruct((64, 1), jnp.float32),
    )(pooled, Wf1, bf1.reshape(1, -1), bn, Wf2, bf2.reshape(1, -1))
    return out[:, 0]


# speedup vs baseline: 3.1342x; 3.1342x over previous
import functools

import jax
import jax.numpy as jnp
import numpy as np
from jax import lax
from jax.experimental import pallas as pl
from jax.experimental.pallas import tpu as pltpu
from jax.experimental.pallas import tpu_sc as plsc

N = 10000
E = 320000
D = 128
G = 64
NC, NS, L = 2, 16, 16
NW = NC * NS
EPW = E // NW
CH = 80
NCHUNK = EPW // CH
NGRP = CH // L
NSH = 10240
NPT = NSH // NS
RB = 1000
SCALE = float(1.0 / np.sqrt(128.0))
NPOOL = 320
NPAD = NW * NPOOL

_mesh = plsc.VectorSubcoreMesh(core_axis_name="c", subcore_axis_name="s",
                               num_cores=NC, num_subcores=NS)
_sc_params = pltpu.CompilerParams(needs_layout_passes=False,
                                  use_tc_tiling_on_sc=False)


def _proj_body(x_ref, w_ref, b_ref, wet_ref, q_ref, k_ref, v_ref, s_ref, qe_ref):
    p = jnp.dot(x_ref[...], w_ref[...], preferred_element_type=jnp.float32)
    p = p + b_ref[...]
    q_ref[...] = p[:, 0:128]
    k_ref[...] = p[:, 128:256]
    v_ref[...] = p[:, 256:384]
    s_ref[...] = p[:, 384:512]
    qe_ref[...] = jnp.dot(p[:, 0:128], wet_ref[...],
                          preferred_element_type=jnp.float32,
                          precision=lax.Precision.HIGHEST)


def _proj(x, wbig, bbig, wet):
    return pl.pallas_call(
        _proj_body,
        grid=(N // RB,),
        in_specs=[
            pl.BlockSpec((RB, 128), lambda i: (i, 0)),
            pl.BlockSpec((128, 512), lambda i: (0, 0)),
            pl.BlockSpec((1, 512), lambda i: (0, 0)),
            pl.BlockSpec((128, 16), lambda i: (0, 0)),
        ],
        out_specs=[
            pl.BlockSpec((RB, 128), lambda i: (i, 0)),
            pl.BlockSpec((RB, 128), lambda i: (i, 0)),
            pl.BlockSpec((RB, 128), lambda i: (i, 0)),
            pl.BlockSpec((RB, 128), lambda i: (i, 0)),
            pl.BlockSpec((RB, 16), lambda i: (i, 0)),
        ],
        out_shape=[
            jax.ShapeDtypeStruct((N, 128), jnp.float32),
            jax.ShapeDtypeStruct((N, 128), jnp.float32),
            jax.ShapeDtypeStruct((N, 128), jnp.float32),
            jax.ShapeDtypeStruct((N, 128), jnp.float32),
            jax.ShapeDtypeStruct((N, 16), jnp.float32),
        ],
    )(x, wbig, bbig, wet)


def _edge_body(q_hbm, k_hbm, v_hbm, qe_hbm, ea_hbm, src_hbm, dst_hbm, z_hbm, z16_hbm,
               agg_out, a14_out,
               srcv, dstv, qrows, krows, vrows, qerows, earows, exv,
               agg_sh, a14_sh, sem):
    cid = lax.axis_index("c")
    sid = lax.axis_index("s")
    wid = sid * NC + cid
    rs = sid * NPT

    pltpu.sync_copy(z_hbm.at[pl.ds(0, NPT), :], agg_sh.at[pl.ds(rs, NPT), :])
    pltpu.sync_copy(z16_hbm.at[pl.ds(0, NPT), :], a14_sh.at[pl.ds(rs, NPT), :])
    plsc.subcore_barrier()

    lanes = lax.iota(jnp.int32, 16)

    def chunk_body(j, _):
        base = wid * EPW + j * CH
        pltpu.sync_copy(src_hbm.at[pl.ds(base, CH)], srcv)
        pltpu.sync_copy(dst_hbm.at[pl.ds(base, CH)], dstv)
        cq = pltpu.async_copy(q_hbm.at[dstv], qrows, sem)
        ck = pltpu.async_copy(k_hbm.at[srcv], krows, sem)
        cv = pltpu.async_copy(v_hbm.at[srcv], vrows, sem)
        ce = pltpu.async_copy(qe_hbm.at[dstv], qerows, sem)
        pltpu.sync_copy(ea_hbm.at[pl.ds(base, CH), :], earows)
        cq.wait()
        ck.wait()
        cv.wait()
        ce.wait()

        def grp_body(g, _):
            rows = g * L + lanes
            acc = jnp.zeros((16,), jnp.float32)
            for dd in range(128):
                col = jnp.full((16,), dd, jnp.int32)
                qv = plsc.load_gather(qrows, [rows, col])
                kv = plsc.load_gather(krows, [rows, col])
                acc = acc + qv * kv
            for dd in range(16):
                col = jnp.full((16,), dd, jnp.int32)
                qev = plsc.load_gather(qerows, [rows, col])
                eav = plsc.load_gather(earows, [rows, col])
                acc = acc + qev * eav
            ex = jnp.exp(acc * SCALE)
            exv[pl.ds(g * L, L)] = ex

            def lane_body(l, _):
                i = g * L + l
                sv = plsc.load_gather(exv, [jnp.full((16,), i, jnp.int32)])
                for c in range(8):
                    vrows[i, pl.ds(c * 16, 16)] = vrows[i, pl.ds(c * 16, 16)] * sv
                earows[i, :] = earows[i, :] * sv
                return 0

            lax.fori_loop(0, L, lane_body, 0)
            return 0

        lax.fori_loop(0, NGRP, grp_body, 0)

        pltpu.sync_copy(vrows, agg_sh.at[dstv], add=True)
        pltpu.sync_copy(earows, a14_sh.at[dstv], add=True)
        return 0

    lax.fori_loop(0, NCHUNK, chunk_body, 0)
    plsc.subcore_barrier()

    pltpu.sync_copy(agg_sh.at[pl.ds(rs, NPT), :], agg_out.at[cid, pl.ds(rs, NPT), :])
    pltpu.sync_copy(a14_sh.at[pl.ds(rs, NPT), :], a14_out.at[cid, pl.ds(rs, NPT), :])


_edge_kernel = pl.kernel(
    _edge_body,
    out_type=[
        jax.ShapeDtypeStruct((NC, NSH, 128), jnp.float32),
        jax.ShapeDtypeStruct((NC, NSH, 16), jnp.float32),
    ],
    mesh=_mesh,
    scratch_types=[
        pltpu.VMEM((CH,), jnp.int32),
        pltpu.VMEM((CH,), jnp.int32),
        pltpu.VMEM((CH, 128), jnp.float32),
        pltpu.VMEM((CH, 128), jnp.float32),
        pltpu.VMEM((CH, 128), jnp.float32),
        pltpu.VMEM((CH, 16), jnp.float32),
        pltpu.VMEM((CH, 16), jnp.float32),
        pltpu.VMEM((CH,), jnp.float32),
        pltpu.VMEM_SHARED((NSH, 128), jnp.float32),
        pltpu.VMEM_SHARED((NSH, 16), jnp.float32),
        pltpu.SemaphoreType.DMA,
    ],
    compiler_params=_sc_params,
)


def _combine(agg, a14, s_ref, wep_ref):
    a14sum = a14[0] + a14[1]
    dent = a14sum[:, 14:15]
    dsafe = jnp.where(dent > 0.0, dent, 1.0)
    aggt = (agg[0] + agg[1]) / dsafe
    a14t = a14sum / dsafe
    h = aggt + jnp.dot(a14t, wep_ref[...], preferred_element_type=jnp.float32,
                       precision=lax.Precision.HIGHEST)
    return jnp.maximum(h + s_ref[...], 0.0)


def _comb_proj_body(agg_ref, a14_ref, s_ref, wep_ref, w_ref, b_ref, wet_ref,
                    q_ref, k_ref, v_ref, s2_ref, qe_ref):
    h = _combine(agg_ref[...], a14_ref[...], s_ref, wep_ref)
    p = jnp.dot(h, w_ref[...], preferred_element_type=jnp.float32) + b_ref[...]
    q_ref[...] = p[:, 0:128]
    k_ref[...] = p[:, 128:256]
    v_ref[...] = p[:, 256:384]
    s2_ref[...] = p[:, 384:512]
    qe_ref[...] = jnp.dot(p[:, 0:128], wet_ref[...],
                          preferred_element_type=jnp.float32,
                          precision=lax.Precision.HIGHEST)


def _comb_proj(aggP, a14P, s, wep, wbig, bbig, wet):
    return pl.pallas_call(
        _comb_proj_body,
        grid=(N // RB,),
        in_specs=[
            pl.BlockSpec((NC, RB, 128), lambda i: (0, i, 0)),
            pl.BlockSpec((NC, RB, 16), lambda i: (0, i, 0)),
            pl.BlockSpec((RB, 128), lambda i: (i, 0)),
            pl.BlockSpec((16, 128), lambda i: (0, 0)),
            pl.BlockSpec((128, 512), lambda i: (0, 0)),
            pl.BlockSpec((1, 512), lambda i: (0, 0)),
            pl.BlockSpec((128, 16), lambda i: (0, 0)),
        ],
        out_specs=[
            pl.BlockSpec((RB, 128), lambda i: (i, 0)),
            pl.BlockSpec((RB, 128), lambda i: (i, 0)),
            pl.BlockSpec((RB, 128), lambda i: (i, 0)),
            pl.BlockSpec((RB, 128), lambda i: (i, 0)),
            pl.BlockSpec((RB, 16), lambda i: (i, 0)),
        ],
        out_shape=[
            jax.ShapeDtypeStruct((N, 128), jnp.float32),
            jax.ShapeDtypeStruct((N, 128), jnp.float32),
            jax.ShapeDtypeStruct((N, 128), jnp.float32),
            jax.ShapeDtypeStruct((N, 128), jnp.float32),
            jax.ShapeDtypeStruct((N, 16), jnp.float32),
        ],
    )(aggP, a14P, s, wep, wbig, bbig, wet)


def _comb_final_body(agg_ref, a14_ref, s_ref, wep_ref, h_ref):
    h_ref[...] = _combine(agg_ref[...], a14_ref[...], s_ref, wep_ref)


def _comb_final(aggP, a14P, s, wep):
    return pl.pallas_call(
        _comb_final_body,
        grid=(N // RB,),
        in_specs=[
            pl.BlockSpec((NC, RB, 128), lambda i: (0, i, 0)),
            pl.BlockSpec((NC, RB, 16), lambda i: (0, i, 0)),
            pl.BlockSpec((RB, 128), lambda i: (i, 0)),
            pl.BlockSpec((16, 128), lambda i: (0, 0)),
        ],
        out_specs=[pl.BlockSpec((RB, 128), lambda i: (i, 0))],
        out_shape=[jax.ShapeDtypeStruct((N, 128), jnp.float32)],
    )(aggP, a14P, s, wep)[0]


def _pool_body(h_hbm, b_hbm, tab_out, batv, hrows, tab, sem):
    cid = lax.axis_index("c")
    sid = lax.axis_index("s")
    wid = sid * NC + cid
    base = wid * NPOOL

    def init_body(g, _):
        for c in range(8):
            tab[g, pl.ds(c * 16, 16)] = jnp.full((16,), -jnp.inf, jnp.float32)
        return 0

    lax.fori_loop(0, G, init_body, 0)

    pltpu.sync_copy(b_hbm.at[pl.ds(base, NPOOL)], batv)
    pltpu.async_copy(h_hbm.at[pl.ds(base, NPOOL), :], hrows, sem).wait()
    lanes = lax.iota(jnp.int32, 16)

    def row_body(r, _):
        bvec = plsc.load_gather(batv, [jnp.full((16,), r, jnp.int32)])
        for c in range(8):
            cols = c * 16 + lanes
            cur = plsc.load_gather(tab, [bvec, cols])
            val = hrows[r, pl.ds(c * 16, 16)]
            plsc.store_scatter(tab, [bvec, cols], jnp.maximum(cur, val))
        return 0

    lax.fori_loop(0, NPOOL, row_body, 0)
    pltpu.sync_copy(tab, tab_out.at[wid])


_pool_kernel = pl.kernel(
    _pool_body,
    out_type=[jax.ShapeDtypeStruct((NW, G, 128), jnp.float32)],
    mesh=_mesh,
    scratch_types=[
        pltpu.VMEM((NPOOL,), jnp.int32),
        pltpu.VMEM((NPOOL, 128), jnp.float32),
        pltpu.VMEM((G, 128), jnp.float32),
        pltpu.SemaphoreType.DMA,
    ],
    compiler_params=_sc_params,
)


def _head_body(tab_ref, wf1_ref, bf1_ref, bn_ref, wf2_ref, bf2_ref, out_ref):
    pooled = jnp.max(tab_ref[...], axis=0)
    pooled = jnp.where(jnp.isfinite(pooled), pooled, 0.0)
    f = jnp.maximum(
        jnp.dot(pooled, wf1_ref[...], preferred_element_type=jnp.float32)
        + bf1_ref[...], 0.0)
    g, b, m, v = (bn_ref[0:1, :], bn_ref[1:2, :], bn_ref[2:3, :], bn_ref[3:4, :])
    f = (f - m) / jnp.sqrt(v + 1e-5) * g + b
    out_ref[...] = jnp.dot(f, wf2_ref[...], preferred_element_type=jnp.float32) + bf2_ref[...]


def _head(tabs, wf1, bf1, bn, wf2, bf2):
    return pl.pallas_call(
        _head_body,
        out_shape=jax.ShapeDtypeStruct((G, 1), jnp.float32),
    )(tabs, wf1, bf1.reshape(1, -1), bn, wf2, bf2.reshape(1, -1))


def _padw(w):
    return jnp.pad(w, ((0, 0), (0, 16 - w.shape[1])))


def _brt(a):
    u = lax.bitcast_convert_type(a, jnp.uint32)
    r = (u + jnp.uint32(0x7FFF) + ((u >> 16) & jnp.uint32(1))) & jnp.uint32(0xFFFF0000)
    return lax.bitcast_convert_type(r, jnp.float32)


def kernel(x, edge_index, edge_attr, batch, Wq1, bq1, Wk1, bk1, Wv1, bv1, We1, Ws1, bs1, Wq2, bq2, Wk2, bk2, Wv2, bv2, We2, Ws2, bs2, Wf1, bf1, bn_g, bn_b, bn_m, bn_v, Wf2, bf2):
    src = edge_index[0].astype(jnp.int32)
    dst = edge_index[1].astype(jnp.int32)
    brt = _brt
    eap = jnp.pad(brt(edge_attr), ((0, 0), (0, 2)))
    eap = eap.at[:, 14].set(1.0)
    zeros = jnp.zeros((NPT, 128), jnp.float32)
    zeros16 = jnp.zeros((NPT, 16), jnp.float32)

    w1 = jnp.concatenate([Wq1, Wk1, Wv1, Ws1], axis=1)
    b1 = jnp.concatenate([bq1, bk1, bv1, bs1]).reshape(1, -1)
    w2 = jnp.concatenate([Wq2, Wk2, Wv2, Ws2], axis=1)
    b2 = jnp.concatenate([bq2, bk2, bv2, bs2]).reshape(1, -1)
    wet1 = _padw(brt(We1).T)
    wet2 = _padw(brt(We2).T)
    we1p = jnp.pad(brt(We1), ((0, 2), (0, 0)))
    we2p = jnp.pad(brt(We2), ((0, 2), (0, 0)))

    q1, k1, v1, s1, qe1 = _proj(x, w1, b1, wet1)
    aggP1, a14P1 = _edge_kernel(q1, k1, v1, qe1, eap, src, dst, zeros, zeros16)
    q2, k2, v2, s2, qe2 = _comb_proj(aggP1, a14P1, s1, we1p, w2, b2, wet2)
    aggP2, a14P2 = _edge_kernel(q2, k2, v2, qe2, eap, src, dst, zeros, zeros16)
    h2 = _comb_final(aggP2, a14P2, s2, we2p)

    h2p = jnp.pad(h2, ((0, NPAD - N), (0, 0)), constant_values=-jnp.inf)
    batp = jnp.pad(batch.astype(jnp.int32), (0, NPAD - N), constant_values=G - 1)
    tabs = _pool_kernel(h2p, batp)[0]

    bn = jnp.stack([bn_g, bn_b, bn_m, bn_v], axis=0)
    out = _head(tabs, Wf1, bf1, bn, Wf2, bf2)
    return out[:, 0]


# speedup vs baseline: 3.1636x; 1.0094x over previous
import functools

import jax
import jax.numpy as jnp
import numpy as np
from jax import lax
from jax.experimental import pallas as pl
from jax.experimental.pallas import tpu as pltpu
from jax.experimental.pallas import tpu_sc as plsc

N = 10000
E = 320000
D = 128
G = 64
NC, NS, L = 2, 16, 16
NW = NC * NS
EPW = E // NW
CH = 80
NCHUNK = EPW // CH
NGRP = CH // L
NSH = 10240
NPT = NSH // NS
RB = 1000
SCALE = float(1.0 / np.sqrt(128.0))
NPOOL = 320
NPAD = NW * NPOOL

_mesh = plsc.VectorSubcoreMesh(core_axis_name="c", subcore_axis_name="s",
                               num_cores=NC, num_subcores=NS)
_sc_params = pltpu.CompilerParams(needs_layout_passes=False,
                                  use_tc_tiling_on_sc=False)


def _proj_body(x_ref, w_ref, b_ref, wet_ref, q_ref, k_ref, v_ref, s_ref, qe_ref):
    p = jnp.dot(x_ref[...], w_ref[...], preferred_element_type=jnp.float32)
    p = p + b_ref[...]
    q_ref[...] = p[:, 0:128]
    k_ref[...] = p[:, 128:256]
    v_ref[...] = p[:, 256:384]
    s_ref[...] = p[:, 384:512]
    qe_ref[...] = jnp.dot(p[:, 0:128], wet_ref[...],
                          preferred_element_type=jnp.float32,
                          precision=lax.Precision.HIGHEST)


def _proj(x, wbig, bbig, wet):
    return pl.pallas_call(
        _proj_body,
        grid=(N // RB,),
        in_specs=[
            pl.BlockSpec((RB, 128), lambda i: (i, 0)),
            pl.BlockSpec((128, 512), lambda i: (0, 0)),
            pl.BlockSpec((1, 512), lambda i: (0, 0)),
            pl.BlockSpec((128, 16), lambda i: (0, 0)),
        ],
        out_specs=[
            pl.BlockSpec((RB, 128), lambda i: (i, 0)),
            pl.BlockSpec((RB, 128), lambda i: (i, 0)),
            pl.BlockSpec((RB, 128), lambda i: (i, 0)),
            pl.BlockSpec((RB, 128), lambda i: (i, 0)),
            pl.BlockSpec((RB, 16), lambda i: (i, 0)),
        ],
        out_shape=[
            jax.ShapeDtypeStruct((N, 128), jnp.float32),
            jax.ShapeDtypeStruct((N, 128), jnp.float32),
            jax.ShapeDtypeStruct((N, 128), jnp.float32),
            jax.ShapeDtypeStruct((N, 128), jnp.float32),
            jax.ShapeDtypeStruct((N, 16), jnp.float32),
        ],
    )(x, wbig, bbig, wet)


def _edge_body(q_hbm, k_hbm, v_hbm, qe_hbm, ea_hbm, src_hbm, dst_hbm, z_hbm, z16_hbm,
               agg_out, a14_out,
               srcv, dstv, qrows, krows, vrows, qerows, earows, exv,
               agg_sh, a14_sh, sem):
    cid = lax.axis_index("c")
    sid = lax.axis_index("s")
    wid = sid * NC + cid
    rs = sid * NPT

    pltpu.sync_copy(z_hbm.at[pl.ds(0, NPT), :], agg_sh.at[pl.ds(rs, NPT), :])
    pltpu.sync_copy(z16_hbm.at[pl.ds(0, NPT), :], a14_sh.at[pl.ds(rs, NPT), :])
    plsc.subcore_barrier()

    lanes = lax.iota(jnp.int32, 16)

    def chunk_body(j, _):
        base = wid * EPW + j * CH
        pltpu.sync_copy(src_hbm.at[pl.ds(base, CH)], srcv)
        pltpu.sync_copy(dst_hbm.at[pl.ds(base, CH)], dstv)
        cq = pltpu.async_copy(q_hbm.at[dstv], qrows, sem)
        ck = pltpu.async_copy(k_hbm.at[srcv], krows, sem)
        cv = pltpu.async_copy(v_hbm.at[srcv], vrows, sem)
        ce = pltpu.async_copy(qe_hbm.at[dstv], qerows, sem)
        pltpu.sync_copy(ea_hbm.at[pl.ds(base, CH), :], earows)
        cq.wait()
        ck.wait()
        cv.wait()
        ce.wait()

        def grp_body(g, _):
            rows = g * L + lanes
            acc = jnp.zeros((16,), jnp.float32)
            for dd in range(128):
                col = jnp.full((16,), dd, jnp.int32)
                qv = plsc.load_gather(qrows, [rows, col])
                kv = plsc.load_gather(krows, [rows, col])
                acc = acc + qv * kv
            for dd in range(16):
                col = jnp.full((16,), dd, jnp.int32)
                qev = plsc.load_gather(qerows, [rows, col])
                eav = plsc.load_gather(earows, [rows, col])
                acc = acc + qev * eav
            ex = jnp.exp(acc * SCALE)
            exv[pl.ds(g * L, L)] = ex

            for l in range(L):
                i = g * L + l
                sv = plsc.load_gather(exv, [jnp.full((16,), i, jnp.int32)])
                for c in range(8):
                    vrows[i, pl.ds(c * 16, 16)] = vrows[i, pl.ds(c * 16, 16)] * sv
                earows[i, :] = earows[i, :] * sv
            return 0

        lax.fori_loop(0, NGRP, grp_body, 0)

        pltpu.sync_copy(vrows, agg_sh.at[dstv], add=True)
        pltpu.sync_copy(earows, a14_sh.at[dstv], add=True)
        return 0

    lax.fori_loop(0, NCHUNK, chunk_body, 0)
    plsc.subcore_barrier()

    pltpu.sync_copy(agg_sh.at[pl.ds(rs, NPT), :], agg_out.at[cid, pl.ds(rs, NPT), :])
    pltpu.sync_copy(a14_sh.at[pl.ds(rs, NPT), :], a14_out.at[cid, pl.ds(rs, NPT), :])


_edge_kernel = pl.kernel(
    _edge_body,
    out_type=[
        jax.ShapeDtypeStruct((NC, NSH, 128), jnp.float32),
        jax.ShapeDtypeStruct((NC, NSH, 16), jnp.float32),
    ],
    mesh=_mesh,
    scratch_types=[
        pltpu.VMEM((CH,), jnp.int32),
        pltpu.VMEM((CH,), jnp.int32),
        pltpu.VMEM((CH, 128), jnp.float32),
        pltpu.VMEM((CH, 128), jnp.float32),
        pltpu.VMEM((CH, 128), jnp.float32),
        pltpu.VMEM((CH, 16), jnp.float32),
        pltpu.VMEM((CH, 16), jnp.float32),
        pltpu.VMEM((CH,), jnp.float32),
        pltpu.VMEM_SHARED((NSH, 128), jnp.float32),
        pltpu.VMEM_SHARED((NSH, 16), jnp.float32),
        pltpu.SemaphoreType.DMA,
    ],
    compiler_params=_sc_params,
)


def _combine(agg, a14, s_ref, wep_ref):
    a14sum = a14[0] + a14[1]
    dent = a14sum[:, 14:15]
    dsafe = jnp.where(dent > 0.0, dent, 1.0)
    aggt = (agg[0] + agg[1]) / dsafe
    a14t = a14sum / dsafe
    h = aggt + jnp.dot(a14t, wep_ref[...], preferred_element_type=jnp.float32,
                       precision=lax.Precision.HIGHEST)
    return jnp.maximum(h + s_ref[...], 0.0)


def _comb_proj_body(agg_ref, a14_ref, s_ref, wep_ref, w_ref, b_ref, wet_ref,
                    q_ref, k_ref, v_ref, s2_ref, qe_ref):
    h = _combine(agg_ref[...], a14_ref[...], s_ref, wep_ref)
    p = jnp.dot(h, w_ref[...], preferred_element_type=jnp.float32) + b_ref[...]
    q_ref[...] = p[:, 0:128]
    k_ref[...] = p[:, 128:256]
    v_ref[...] = p[:, 256:384]
    s2_ref[...] = p[:, 384:512]
    qe_ref[...] = jnp.dot(p[:, 0:128], wet_ref[...],
                          preferred_element_type=jnp.float32,
                          precision=lax.Precision.HIGHEST)


def _comb_proj(aggP, a14P, s, wep, wbig, bbig, wet):
    return pl.pallas_call(
        _comb_proj_body,
        grid=(N // RB,),
        in_specs=[
            pl.BlockSpec((NC, RB, 128), lambda i: (0, i, 0)),
            pl.BlockSpec((NC, RB, 16), lambda i: (0, i, 0)),
            pl.BlockSpec((RB, 128), lambda i: (i, 0)),
            pl.BlockSpec((16, 128), lambda i: (0, 0)),
            pl.BlockSpec((128, 512), lambda i: (0, 0)),
            pl.BlockSpec((1, 512), lambda i: (0, 0)),
            pl.BlockSpec((128, 16), lambda i: (0, 0)),
        ],
        out_specs=[
            pl.BlockSpec((RB, 128), lambda i: (i, 0)),
            pl.BlockSpec((RB, 128), lambda i: (i, 0)),
            pl.BlockSpec((RB, 128), lambda i: (i, 0)),
            pl.BlockSpec((RB, 128), lambda i: (i, 0)),
            pl.BlockSpec((RB, 16), lambda i: (i, 0)),
        ],
        out_shape=[
            jax.ShapeDtypeStruct((N, 128), jnp.float32),
            jax.ShapeDtypeStruct((N, 128), jnp.float32),
            jax.ShapeDtypeStruct((N, 128), jnp.float32),
            jax.ShapeDtypeStruct((N, 128), jnp.float32),
            jax.ShapeDtypeStruct((N, 16), jnp.float32),
        ],
    )(aggP, a14P, s, wep, wbig, bbig, wet)


def _comb_final_body(agg_ref, a14_ref, s_ref, wep_ref, h_ref):
    h_ref[...] = _combine(agg_ref[...], a14_ref[...], s_ref, wep_ref)


def _comb_final(aggP, a14P, s, wep):
    return pl.pallas_call(
        _comb_final_body,
        grid=(N // RB,),
        in_specs=[
            pl.BlockSpec((NC, RB, 128), lambda i: (0, i, 0)),
            pl.BlockSpec((NC, RB, 16), lambda i: (0, i, 0)),
            pl.BlockSpec((RB, 128), lambda i: (i, 0)),
            pl.BlockSpec((16, 128), lambda i: (0, 0)),
        ],
        out_specs=[pl.BlockSpec((RB, 128), lambda i: (i, 0))],
        out_shape=[jax.ShapeDtypeStruct((N, 128), jnp.float32)],
    )(aggP, a14P, s, wep)[0]


def _pool_body(h_hbm, b_hbm, tab_out, batv, hrows, tab, sem):
    cid = lax.axis_index("c")
    sid = lax.axis_index("s")
    wid = sid * NC + cid
    base = wid * NPOOL

    def init_body(g, _):
        for c in range(8):
            tab[g, pl.ds(c * 16, 16)] = jnp.full((16,), -jnp.inf, jnp.float32)
        return 0

    lax.fori_loop(0, G, init_body, 0)

    pltpu.sync_copy(b_hbm.at[pl.ds(base, NPOOL)], batv)
    pltpu.async_copy(h_hbm.at[pl.ds(base, NPOOL), :], hrows, sem).wait()
    lanes = lax.iota(jnp.int32, 16)

    def row_body(r, _):
        bvec = plsc.load_gather(batv, [jnp.full((16,), r, jnp.int32)])
        for c in range(8):
            cols = c * 16 + lanes
            cur = plsc.load_gather(tab, [bvec, cols])
            val = hrows[r, pl.ds(c * 16, 16)]
            plsc.store_scatter(tab, [bvec, cols], jnp.maximum(cur, val))
        return 0

    lax.fori_loop(0, NPOOL, row_body, 0)
    pltpu.sync_copy(tab, tab_out.at[wid])


_pool_kernel = pl.kernel(
    _pool_body,
    out_type=[jax.ShapeDtypeStruct((NW, G, 128), jnp.float32)],
    mesh=_mesh,
    scratch_types=[
        pltpu.VMEM((NPOOL,), jnp.int32),
        pltpu.VMEM((NPOOL, 128), jnp.float32),
        pltpu.VMEM((G, 128), jnp.float32),
        pltpu.SemaphoreType.DMA,
    ],
    compiler_params=_sc_params,
)


def _head_body(tab_ref, wf1_ref, bf1_ref, bn_ref, wf2_ref, bf2_ref, out_ref):
    pooled = jnp.max(tab_ref[...], axis=0)
    pooled = jnp.where(jnp.isfinite(pooled), pooled, 0.0)
    f = jnp.maximum(
        jnp.dot(pooled, wf1_ref[...], preferred_element_type=jnp.float32)
        + bf1_ref[...], 0.0)
    g, b, m, v = (bn_ref[0:1, :], bn_ref[1:2, :], bn_ref[2:3, :], bn_ref[3:4, :])
    f = (f - m) / jnp.sqrt(v + 1e-5) * g + b
    out_ref[...] = jnp.dot(f, wf2_ref[...], preferred_element_type=jnp.float32) + bf2_ref[...]


def _head(tabs, wf1, bf1, bn, wf2, bf2):
    return pl.pallas_call(
        _head_body,
        out_shape=jax.ShapeDtypeStruct((G, 1), jnp.float32),
    )(tabs, wf1, bf1.reshape(1, -1), bn, wf2, bf2.reshape(1, -1))


def _padw(w):
    return jnp.pad(w, ((0, 0), (0, 16 - w.shape[1])))


def _brt(a):
    u = lax.bitcast_convert_type(a, jnp.uint32)
    r = (u + jnp.uint32(0x7FFF) + ((u >> 16) & jnp.uint32(1))) & jnp.uint32(0xFFFF0000)
    return lax.bitcast_convert_type(r, jnp.float32)


def kernel(x, edge_index, edge_attr, batch, Wq1, bq1, Wk1, bk1, Wv1, bv1, We1, Ws1, bs1, Wq2, bq2, Wk2, bk2, Wv2, bv2, We2, Ws2, bs2, Wf1, bf1, bn_g, bn_b, bn_m, bn_v, Wf2, bf2):
    src = edge_index[0].astype(jnp.int32)
    dst = edge_index[1].astype(jnp.int32)
    brt = _brt
    eap = jnp.pad(brt(edge_attr), ((0, 0), (0, 2)))
    eap = eap.at[:, 14].set(1.0)
    zeros = jnp.zeros((NPT, 128), jnp.float32)
    zeros16 = jnp.zeros((NPT, 16), jnp.float32)

    w1 = jnp.concatenate([Wq1, Wk1, Wv1, Ws1], axis=1)
    b1 = jnp.concatenate([bq1, bk1, bv1, bs1]).reshape(1, -1)
    w2 = jnp.concatenate([Wq2, Wk2, Wv2, Ws2], axis=1)
    b2 = jnp.concatenate([bq2, bk2, bv2, bs2]).reshape(1, -1)
    wet1 = _padw(brt(We1).T)
    wet2 = _padw(brt(We2).T)
    we1p = jnp.pad(brt(We1), ((0, 2), (0, 0)))
    we2p = jnp.pad(brt(We2), ((0, 2), (0, 0)))

    q1, k1, v1, s1, qe1 = _proj(x, w1, b1, wet1)
    aggP1, a14P1 = _edge_kernel(q1, k1, v1, qe1, eap, src, dst, zeros, zeros16)
    q2, k2, v2, s2, qe2 = _comb_proj(aggP1, a14P1, s1, we1p, w2, b2, wet2)
    aggP2, a14P2 = _edge_kernel(q2, k2, v2, qe2, eap, src, dst, zeros, zeros16)
    h2 = _comb_final(aggP2, a14P2, s2, we2p)

    h2p = jnp.pad(h2, ((0, NPAD - N), (0, 0)), constant_values=-jnp.inf)
    batp = jnp.pad(batch.astype(jnp.int32), (0, NPAD - N), constant_values=G - 1)
    tabs = _pool_kernel(h2p, batp)[0]

    bn = jnp.stack([bn_g, bn_b, bn_m, bn_v], axis=0)
    out = _head(tabs, Wf1, bf1, bn, Wf2, bf2)
    return out[:, 0]
